# Initial kernel scaffold; baseline (speedup 1.0000x reference)
import jax
import jax.numpy as jnp
from jax.experimental import pallas as pl


def kernel(nodes, mask, none_selected, edge_index, W1, W2, W3, ln_scale, ln_bias):
    raise NotImplementedError("write your pallas kernel here")


# speedup vs baseline: 17.2616x; 17.2616x over previous
import functools

import jax
import jax.numpy as jnp
from jax import lax
from jax.experimental import pallas as pl
from jax.experimental.pallas import tpu as pltpu
from jax.experimental.pallas import tpu_sc as plsc

N = 10000
E = 320000
D = 128
K = 8
FW = 2 * K

NSUB = 16
NP = 10240
ROWS_PT = NP // NSUB
EDG_PT = E // NSUB
CH = 2000
NCH = EDG_PT // CH


def _project_body(nodes_ref, mask_ref, w1_ref, w23_ref, out_ref):
    t = nodes_ref[...] * mask_ref[...]
    h = jnp.dot(t, w1_ref[...], preferred_element_type=jnp.float32)
    out_ref[...] = jnp.dot(h, w23_ref[...], preferred_element_type=jnp.float32)


def _project(nodes, mask, W1, W23):
    TN = 2000
    return pl.pallas_call(
        _project_body,
        grid=(N // TN,),
        in_specs=[
            pl.BlockSpec((TN, D), lambda i: (i, 0)),
            pl.BlockSpec((TN, 1), lambda i: (i, 0)),
            pl.BlockSpec((D, D), lambda i: (0, 0)),
            pl.BlockSpec((D, FW), lambda i: (0, 0)),
        ],
        out_specs=pl.BlockSpec((TN, FW), lambda i: (i, 0)),
        out_shape=jax.ShapeDtypeStruct((N, FW), jnp.float32),
    )(nodes, mask, W1, W23)


def _sc_body(src_hbm, dst_hbm, m_hbm,
             z_hbm, c_hbm, d_hbm,
             s_sh, g_sh, c_sh,
             idx_s, idx_d, rows, ones, sbuf, dbuf, sem):
    tid = lax.axis_index("s")
    nbase = tid * ROWS_PT

    def _zero_row(i, _):
        sbuf[i, :] = jnp.zeros((FW,), jnp.float32)
        return 0
    lax.fori_loop(0, ROWS_PT, _zero_row, 0)

    def _one_row(i, _):
        ones[i, :] = jnp.ones((FW,), jnp.float32)
        return 0
    lax.fori_loop(0, CH, _one_row, 0)

    pltpu.sync_copy(sbuf, s_sh.at[pl.ds(nbase, ROWS_PT)])
    pltpu.sync_copy(sbuf, g_sh.at[pl.ds(nbase, ROWS_PT)])
    pltpu.sync_copy(sbuf, c_sh.at[pl.ds(nbase, ROWS_PT)])
    plsc.subcore_barrier()

    def _round1(ch, _):
        base = tid * EDG_PT + ch * CH
        pltpu.sync_copy(src_hbm.at[pl.ds(base, CH)], idx_s)
        pltpu.sync_copy(dst_hbm.at[pl.ds(base, CH)], idx_d)
        pltpu.async_copy(m_hbm.at[idx_s], rows, sem).wait()
        pltpu.sync_copy(rows, s_sh.at[idx_d], add=True)
        pltpu.sync_copy(ones, g_sh.at[idx_d], add=True)
        return 0
    lax.fori_loop(0, NCH, _round1, 0)
    plsc.subcore_barrier()

    pltpu.sync_copy(s_sh.at[pl.ds(nbase, ROWS_PT)], sbuf)
    pltpu.sync_copy(g_sh.at[pl.ds(nbase, ROWS_PT)], dbuf)

    def _div(i, _):
        sbuf[i, :] = sbuf[i, :] / jnp.maximum(dbuf[i, :], 1.0)
        return 0
    lax.fori_loop(0, ROWS_PT, _div, 0)
    pltpu.sync_copy(sbuf, z_hbm.at[pl.ds(nbase, ROWS_PT)])
    pltpu.sync_copy(dbuf, d_hbm.at[pl.ds(nbase, ROWS_PT)])
    plsc.subcore_barrier()

    def _round2(ch, _):
        base = tid * EDG_PT + ch * CH
        pltpu.sync_copy(src_hbm.at[pl.ds(base, CH)], idx_s)
        pltpu.sync_copy(dst_hbm.at[pl.ds(base, CH)], idx_d)
        pltpu.async_copy(z_hbm.at[idx_s], rows, sem).wait()
        pltpu.sync_copy(rows, c_sh.at[idx_d], add=True)
        return 0
    lax.fori_loop(0, NCH, _round2, 0)
    plsc.subcore_barrier()

    pltpu.sync_copy(c_sh.at[pl.ds(nbase, ROWS_PT)], sbuf)
    pltpu.sync_copy(sbuf, c_hbm.at[pl.ds(nbase, ROWS_PT)])


def _sc_passes(src, dst, M):
    mesh = plsc.VectorSubcoreMesh(
        core_axis_name="c", subcore_axis_name="s", num_cores=1)
    f = functools.partial(
        pl.kernel,
        out_type=(
            jax.ShapeDtypeStruct((NP, FW), jnp.float32),
            jax.ShapeDtypeStruct((NP, FW), jnp.float32),
            jax.ShapeDtypeStruct((NP, FW), jnp.float32),
        ),
        mesh=mesh,
        scratch_types=[
            pltpu.VMEM_SHARED((NP, FW), jnp.float32),
            pltpu.VMEM_SHARED((NP, FW), jnp.float32),
            pltpu.VMEM_SHARED((NP, FW), jnp.float32),
            pltpu.VMEM((CH,), jnp.int32),
            pltpu.VMEM((CH,), jnp.int32),
            pltpu.VMEM((CH, FW), jnp.float32),
            pltpu.VMEM((CH, FW), jnp.float32),
            pltpu.VMEM((ROWS_PT, FW), jnp.float32),
            pltpu.VMEM((ROWS_PT, FW), jnp.float32),
            pltpu.SemaphoreType.DMA,
        ],
        compiler_params=pltpu.CompilerParams(use_tc_tiling_on_sc=False),
    )(_sc_body)
    return f(src, dst, M)


def _epilogue_body(c_ref, z_ref, g_ref, sc_ref, b_ref, out_ref):
    deg = jnp.maximum(g_ref[...], 1.0)
    x = c_ref[...] / deg + z_ref[...]
    mu = jnp.mean(x, axis=-1, keepdims=True)
    var = jnp.mean((x - mu) ** 2, axis=-1, keepdims=True)
    y = (x - mu) * lax.rsqrt(var + 1e-5) * sc_ref[...] + b_ref[...]
    out_ref[...] = jnp.maximum(y, 0.0)


def _epilogue(c8, z3, degc, ln_scale, ln_bias):
    TN = 2000
    return pl.pallas_call(
        _epilogue_body,
        grid=(N // TN,),
        in_specs=[
            pl.BlockSpec((TN, K), lambda i: (i, 0)),
            pl.BlockSpec((TN, K), lambda i: (i, 0)),
            pl.BlockSpec((TN, 1), lambda i: (i, 0)),
            pl.BlockSpec((1, K), lambda i: (0, 0)),
            pl.BlockSpec((1, K), lambda i: (0, 0)),
        ],
        out_specs=pl.BlockSpec((TN, K), lambda i: (i, 0)),
        out_shape=jax.ShapeDtypeStruct((N, K), jnp.float32),
    )(c8, z3, degc, ln_scale, ln_bias)


def kernel(nodes, mask, none_selected, edge_index, W1, W2, W3, ln_scale, ln_bias):
    del none_selected
    W23 = jnp.concatenate([W2, W3], axis=1)
    M = _project(nodes, mask, W1, W23)
    Mp = jnp.zeros((NP, FW), jnp.float32).at[:N].set(M)
    src = edge_index[0]
    dst = edge_index[1]
    Z, C, deg = _sc_passes(src, dst, Mp)
    out = _epilogue(C[:N, :K], Z[:N, K:], deg[:N, :1],
                    ln_scale.reshape(1, K), ln_bias.reshape(1, K))
    return out


# speedup vs baseline: 21.1683x; 1.2263x over previous
import functools

import jax
import jax.numpy as jnp
from jax import lax
from jax.experimental import pallas as pl
from jax.experimental.pallas import tpu as pltpu
from jax.experimental.pallas import tpu_sc as plsc

N = 10000
E = 320000
D = 128
K = 8
FW = 2 * K

NSUB = 16
NP = 10240
ROWS_PT = NP // NSUB
EDG_PT = E // NSUB
CH = 1000
NCH = EDG_PT // CH


def _project_body(nodes_ref, mask_ref, w1_ref, w23_ref, out_ref):
    t = nodes_ref[...] * mask_ref[...]
    h = jnp.dot(t, w1_ref[...], preferred_element_type=jnp.float32,
                precision=lax.Precision.HIGHEST)
    out_ref[...] = jnp.dot(h, w23_ref[...], preferred_element_type=jnp.float32,
                           precision=lax.Precision.HIGHEST)


def _project(nodes, mask, W1, W23):
    TN = 2000
    return pl.pallas_call(
        _project_body,
        grid=(N // TN,),
        in_specs=[
            pl.BlockSpec((TN, D), lambda i: (i, 0)),
            pl.BlockSpec((TN, 1), lambda i: (i, 0)),
            pl.BlockSpec((D, D), lambda i: (0, 0)),
            pl.BlockSpec((D, FW), lambda i: (0, 0)),
        ],
        out_specs=pl.BlockSpec((TN, FW), lambda i: (i, 0)),
        out_shape=jax.ShapeDtypeStruct((NP, FW), jnp.float32),
    )(nodes, mask, W1, W23)


def _sc_body(src_hbm, dst_hbm, m_hbm,
             z_hbm, c_hbm, d_hbm,
             s_sh, g_sh, c_sh,
             idx_s0, idx_s1, idx_d0, idx_d1, rows0, rows1, ones,
             si0, si1, sd0, sd1, sg0, sg1):
    tid = lax.axis_index("s")
    nbase = tid * ROWS_PT
    ebase = tid * EDG_PT
    idx_s = (idx_s0, idx_s1)
    idx_d = (idx_d0, idx_d1)
    rows = (rows0, rows1)
    si = (si0, si1)
    sd = (sd0, sd1)
    sg = (sg0, sg1)

    def _zero_row(i, _):
        rows0[i, :] = jnp.zeros((FW,), jnp.float32)
        return 0
    lax.fori_loop(0, ROWS_PT, _zero_row, 0)

    def _one_row(i, _):
        ones[i, :] = jnp.ones((FW,), jnp.float32)
        return 0
    lax.fori_loop(0, CH, _one_row, 0)

    zsrc = rows0.at[pl.ds(0, ROWS_PT)]
    pltpu.sync_copy(zsrc, s_sh.at[pl.ds(nbase, ROWS_PT)])
    pltpu.sync_copy(zsrc, g_sh.at[pl.ds(nbase, ROWS_PT)])
    pltpu.sync_copy(zsrc, c_sh.at[pl.ds(nbase, ROWS_PT)])
    plsc.subcore_barrier()

    def _pipelined_round(gather_src, do_round1):
        def _issue_idx(k):
            b = k % 2
            base = ebase + k * CH
            pltpu.async_copy(src_hbm.at[pl.ds(base, CH)], idx_s[b], si[b])
            pltpu.async_copy(dst_hbm.at[pl.ds(base, CH)], idx_d[b], sd[b])

        def _issue_gather(k):
            b = k % 2
            pltpu.make_async_copy(src_hbm.at[pl.ds(ebase, CH)],
                                  idx_s[b], si[b]).wait()
            pltpu.async_copy(gather_src.at[idx_s[b]], rows[b], sg[b])

        _issue_idx(0)
        _issue_gather(0)
        _issue_idx(1)
        for k in range(NCH):
            b = k % 2
            if k + 1 < NCH:
                _issue_gather(k + 1)
            pltpu.make_async_copy(gather_src.at[idx_s[b]],
                                  rows[b], sg[b]).wait()
            pltpu.make_async_copy(dst_hbm.at[pl.ds(ebase, CH)],
                                  idx_d[b], sd[b]).wait()
            if do_round1:
                pltpu.sync_copy(rows[b], s_sh.at[idx_d[b]], add=True)
                pltpu.sync_copy(ones, g_sh.at[idx_d[b]], add=True)
            else:
                pltpu.sync_copy(rows[b], c_sh.at[idx_d[b]], add=True)
            if k + 2 < NCH:
                _issue_idx(k + 2)

    _pipelined_round(m_hbm, True)
    plsc.subcore_barrier()

    sbuf = rows0.at[pl.ds(0, ROWS_PT)]
    dbuf = ones.at[pl.ds(0, ROWS_PT)]
    pltpu.sync_copy(s_sh.at[pl.ds(nbase, ROWS_PT)], sbuf)
    pltpu.sync_copy(g_sh.at[pl.ds(nbase, ROWS_PT)], dbuf)

    def _div(i, _):
        rows0[i, :] = rows0[i, :] / jnp.maximum(ones[i, :], 1.0)
        return 0
    lax.fori_loop(0, ROWS_PT, _div, 0)
    pltpu.sync_copy(sbuf, z_hbm.at[pl.ds(nbase, ROWS_PT)])
    pltpu.sync_copy(dbuf, d_hbm.at[pl.ds(nbase, ROWS_PT)])
    plsc.subcore_barrier()

    _pipelined_round(z_hbm, False)
    plsc.subcore_barrier()

    cbuf = rows0.at[pl.ds(0, ROWS_PT)]
    pltpu.sync_copy(c_sh.at[pl.ds(nbase, ROWS_PT)], cbuf)
    pltpu.sync_copy(cbuf, c_hbm.at[pl.ds(nbase, ROWS_PT)])


def _sc_passes(src, dst, M):
    mesh = plsc.VectorSubcoreMesh(
        core_axis_name="c", subcore_axis_name="s", num_cores=1)
    f = functools.partial(
        pl.kernel,
        out_type=(
            jax.ShapeDtypeStruct((NP, FW), jnp.float32),
            jax.ShapeDtypeStruct((NP, FW), jnp.float32),
            jax.ShapeDtypeStruct((NP, FW), jnp.float32),
        ),
        mesh=mesh,
        scratch_types=[
            pltpu.VMEM_SHARED((NP, FW), jnp.float32),
            pltpu.VMEM_SHARED((NP, FW), jnp.float32),
            pltpu.VMEM_SHARED((NP, FW), jnp.float32),
            pltpu.VMEM((CH,), jnp.int32),
            pltpu.VMEM((CH,), jnp.int32),
            pltpu.VMEM((CH,), jnp.int32),
            pltpu.VMEM((CH,), jnp.int32),
            pltpu.VMEM((CH, FW), jnp.float32),
            pltpu.VMEM((CH, FW), jnp.float32),
            pltpu.VMEM((CH, FW), jnp.float32),
            pltpu.SemaphoreType.DMA,
            pltpu.SemaphoreType.DMA,
            pltpu.SemaphoreType.DMA,
            pltpu.SemaphoreType.DMA,
            pltpu.SemaphoreType.DMA,
            pltpu.SemaphoreType.DMA,
        ],
        compiler_params=pltpu.CompilerParams(use_tc_tiling_on_sc=False),
    )(_sc_body)
    return f(src, dst, M)


def _epilogue_body(c_ref, z_ref, g_ref, sc_ref, b_ref, out_ref):
    deg = jnp.maximum(g_ref[...], 1.0)
    x = c_ref[...] / deg + z_ref[...]
    mu = jnp.mean(x, axis=-1, keepdims=True)
    var = jnp.mean((x - mu) ** 2, axis=-1, keepdims=True)
    y = (x - mu) * lax.rsqrt(var + 1e-5) * sc_ref[...] + b_ref[...]
    out_ref[...] = jnp.maximum(y, 0.0)


def _epilogue(c8, z3, degc, ln_scale, ln_bias):
    TN = 2000
    return pl.pallas_call(
        _epilogue_body,
        grid=(N // TN,),
        in_specs=[
            pl.BlockSpec((TN, K), lambda i: (i, 0)),
            pl.BlockSpec((TN, K), lambda i: (i, 0)),
            pl.BlockSpec((TN, 1), lambda i: (i, 0)),
            pl.BlockSpec((1, K), lambda i: (0, 0)),
            pl.BlockSpec((1, K), lambda i: (0, 0)),
        ],
        out_specs=pl.BlockSpec((TN, K), lambda i: (i, 0)),
        out_shape=jax.ShapeDtypeStruct((N, K), jnp.float32),
    )(c8, z3, degc, ln_scale, ln_bias)


def kernel(nodes, mask, none_selected, edge_index, W1, W2, W3, ln_scale, ln_bias):
    del none_selected
    W23 = jnp.concatenate([W2, W3], axis=1)
    M = _project(nodes, mask, W1, W23)
    src = edge_index[0]
    dst = edge_index[1]
    Z, C, deg = _sc_passes(src, dst, M)
    out = _epilogue(C[:N, :K], Z[:N, K:], deg[:N, :1],
                    ln_scale.reshape(1, K), ln_bias.reshape(1, K))
    return out


# speedup vs baseline: 24.8255x; 1.1728x over previous
import functools

import jax
import jax.numpy as jnp
from jax import lax
from jax.experimental import pallas as pl
from jax.experimental.pallas import tpu as pltpu
from jax.experimental.pallas import tpu_sc as plsc

N = 10000
E = 320000
D = 128
K = 8
FW = 2 * K

NSUB = 16
NP = 10240
ROWS_PT = NP // NSUB
EDG_PT = E // NSUB
CH = 1000
NCH = EDG_PT // CH


def _project_body(nodes_ref, mask_ref, w1_ref, w23_ref, out_ref):
    t = nodes_ref[...] * mask_ref[...]
    h = jnp.dot(t, w1_ref[...], preferred_element_type=jnp.float32)
    out_ref[...] = jnp.dot(h, w23_ref[...], preferred_element_type=jnp.float32)


def _project(nodes, mask, W1, W23):
    TN = 2000
    return pl.pallas_call(
        _project_body,
        grid=(N // TN,),
        in_specs=[
            pl.BlockSpec((TN, D), lambda i: (i, 0)),
            pl.BlockSpec((TN, 1), lambda i: (i, 0)),
            pl.BlockSpec((D, D), lambda i: (0, 0)),
            pl.BlockSpec((D, FW), lambda i: (0, 0)),
        ],
        out_specs=pl.BlockSpec((TN, FW), lambda i: (i, 0)),
        out_shape=jax.ShapeDtypeStruct((NP, FW), jnp.float32),
    )(nodes, mask, W1, W23)


def _sc_body(ei_hbm, m_hbm,
             z_hbm, c_hbm, d_hbm,
             s_sh, g_sh, c_sh,
             idx_s0, idx_s1, idx_d0, idx_d1, rows0, rows1, ones,
             si0, si1, sd0, sd1, sg0, sg1):
    tid = lax.axis_index("s")
    nbase = tid * ROWS_PT
    ebase = tid * EDG_PT
    idx_s = (idx_s0, idx_s1)
    idx_d = (idx_d0, idx_d1)
    rows = (rows0, rows1)
    si = (si0, si1)
    sd = (sd0, sd1)
    sg = (sg0, sg1)

    def _zero_row(i, _):
        rows0[i, :] = jnp.zeros((FW,), jnp.float32)
        return 0
    lax.fori_loop(0, ROWS_PT, _zero_row, 0)

    def _one_row(i, _):
        ones[i, :] = jnp.ones((FW,), jnp.float32)
        return 0
    lax.fori_loop(0, CH, _one_row, 0)

    zsrc = rows0.at[pl.ds(0, ROWS_PT)]
    pltpu.sync_copy(zsrc, s_sh.at[pl.ds(nbase, ROWS_PT)])
    pltpu.sync_copy(zsrc, g_sh.at[pl.ds(nbase, ROWS_PT)])
    pltpu.sync_copy(zsrc, c_sh.at[pl.ds(nbase, ROWS_PT)])
    plsc.subcore_barrier()

    def _pipelined_round(gather_src, do_round1):
        def _issue_idx(k):
            b = k % 2
            base = ebase + k * CH
            pltpu.async_copy(ei_hbm.at[0, pl.ds(base, CH)], idx_s[b], si[b])
            pltpu.async_copy(ei_hbm.at[1, pl.ds(base, CH)], idx_d[b], sd[b])

        def _issue_gather(k):
            b = k % 2
            pltpu.make_async_copy(ei_hbm.at[0, pl.ds(ebase, CH)],
                                  idx_s[b], si[b]).wait()
            pltpu.async_copy(gather_src.at[idx_s[b]], rows[b], sg[b])

        _issue_idx(0)
        _issue_gather(0)
        _issue_idx(1)
        for k in range(NCH):
            b = k % 2
            if k + 1 < NCH:
                _issue_gather(k + 1)
            pltpu.make_async_copy(gather_src.at[idx_s[b]],
                                  rows[b], sg[b]).wait()
            pltpu.make_async_copy(ei_hbm.at[1, pl.ds(ebase, CH)],
                                  idx_d[b], sd[b]).wait()
            if do_round1:
                pltpu.sync_copy(rows[b], s_sh.at[idx_d[b]], add=True)
                pltpu.sync_copy(ones, g_sh.at[idx_d[b]], add=True)
            else:
                pltpu.sync_copy(rows[b], c_sh.at[idx_d[b]], add=True)
            if k + 2 < NCH:
                _issue_idx(k + 2)

    _pipelined_round(m_hbm, True)
    plsc.subcore_barrier()

    sbuf = rows0.at[pl.ds(0, ROWS_PT)]
    dbuf = ones.at[pl.ds(0, ROWS_PT)]
    pltpu.sync_copy(s_sh.at[pl.ds(nbase, ROWS_PT)], sbuf)
    pltpu.sync_copy(g_sh.at[pl.ds(nbase, ROWS_PT)], dbuf)

    def _div(i, _):
        rows0[i, :] = rows0[i, :] / jnp.maximum(ones[i, :], 1.0)
        return 0
    lax.fori_loop(0, ROWS_PT, _div, 0)
    pltpu.sync_copy(sbuf, z_hbm.at[pl.ds(nbase, ROWS_PT)])
    pltpu.sync_copy(dbuf, d_hbm.at[pl.ds(nbase, ROWS_PT)])
    plsc.subcore_barrier()

    _pipelined_round(z_hbm, False)
    plsc.subcore_barrier()

    cbuf = rows0.at[pl.ds(0, ROWS_PT)]
    pltpu.sync_copy(c_sh.at[pl.ds(nbase, ROWS_PT)], cbuf)
    pltpu.sync_copy(cbuf, c_hbm.at[pl.ds(nbase, ROWS_PT)])


def _sc_passes(edge_index, M):
    mesh = plsc.VectorSubcoreMesh(
        core_axis_name="c", subcore_axis_name="s", num_cores=1)
    f = functools.partial(
        pl.kernel,
        out_type=(
            jax.ShapeDtypeStruct((NP, FW), jnp.float32),
            jax.ShapeDtypeStruct((NP, FW), jnp.float32),
            jax.ShapeDtypeStruct((NP, FW), jnp.float32),
        ),
        mesh=mesh,
        scratch_types=[
            pltpu.VMEM_SHARED((NP, FW), jnp.float32),
            pltpu.VMEM_SHARED((NP, FW), jnp.float32),
            pltpu.VMEM_SHARED((NP, FW), jnp.float32),
            pltpu.VMEM((CH,), jnp.int32),
            pltpu.VMEM((CH,), jnp.int32),
            pltpu.VMEM((CH,), jnp.int32),
            pltpu.VMEM((CH,), jnp.int32),
            pltpu.VMEM((CH, FW), jnp.float32),
            pltpu.VMEM((CH, FW), jnp.float32),
            pltpu.VMEM((CH, FW), jnp.float32),
            pltpu.SemaphoreType.DMA,
            pltpu.SemaphoreType.DMA,
            pltpu.SemaphoreType.DMA,
            pltpu.SemaphoreType.DMA,
            pltpu.SemaphoreType.DMA,
            pltpu.SemaphoreType.DMA,
        ],
        compiler_params=pltpu.CompilerParams(use_tc_tiling_on_sc=False),
    )(_sc_body)
    return f(edge_index, M)


def _epilogue_body(c_ref, z_ref, g_ref, sc_ref, b_ref, out_ref):
    deg = jnp.maximum(g_ref[...][:, :1], 1.0)
    x = c_ref[...][:, :K] / deg + z_ref[...][:, K:]
    mu = jnp.mean(x, axis=-1, keepdims=True)
    var = jnp.mean((x - mu) ** 2, axis=-1, keepdims=True)
    y = (x - mu) * lax.rsqrt(var + 1e-5) * sc_ref[...] + b_ref[...]
    out_ref[...] = jnp.maximum(y, 0.0)


def _epilogue(C, Z, deg, ln_scale, ln_bias):
    TN = 2000
    return pl.pallas_call(
        _epilogue_body,
        grid=(N // TN,),
        in_specs=[
            pl.BlockSpec((TN, FW), lambda i: (i, 0)),
            pl.BlockSpec((TN, FW), lambda i: (i, 0)),
            pl.BlockSpec((TN, FW), lambda i: (i, 0)),
            pl.BlockSpec((1, K), lambda i: (0, 0)),
            pl.BlockSpec((1, K), lambda i: (0, 0)),
        ],
        out_specs=pl.BlockSpec((TN, K), lambda i: (i, 0)),
        out_shape=jax.ShapeDtypeStruct((N, K), jnp.float32),
    )(C, Z, deg, ln_scale, ln_bias)


def kernel(nodes, mask, none_selected, edge_index, W1, W2, W3, ln_scale, ln_bias):
    del none_selected
    W23 = jnp.concatenate([W2, W3], axis=1)
    M = _project(nodes, mask, W1, W23)
    Z, C, deg = _sc_passes(edge_index, M)
    out = _epilogue(C, Z, deg,
                    ln_scale.reshape(1, K), ln_bias.reshape(1, K))
    return out


# speedup vs baseline: 25.6164x; 1.0319x over previous
import functools

import jax
import jax.numpy as jnp
from jax import lax
from jax.experimental import pallas as pl
from jax.experimental.pallas import tpu as pltpu
from jax.experimental.pallas import tpu_sc as plsc

N = 10000
E = 320000
D = 128
K = 8
FW = 2 * K

NSUB = 16
NP = 10240
ROWS_PT = NP // NSUB
EDG_PT = E // NSUB
CH = 2000
NCH = EDG_PT // CH


def _project_body(nodes_ref, mask_ref, w1_ref, w23_ref, out_ref):
    t = nodes_ref[...] * mask_ref[...]
    h = jnp.dot(t, w1_ref[...], preferred_element_type=jnp.float32)
    out_ref[...] = jnp.dot(h, w23_ref[...], preferred_element_type=jnp.float32)


def _project(nodes, mask, W1, W23):
    TN = 2000
    return pl.pallas_call(
        _project_body,
        grid=(N // TN,),
        in_specs=[
            pl.BlockSpec((TN, D), lambda i: (i, 0)),
            pl.BlockSpec((TN, 1), lambda i: (i, 0)),
            pl.BlockSpec((D, D), lambda i: (0, 0)),
            pl.BlockSpec((D, FW), lambda i: (0, 0)),
        ],
        out_specs=pl.BlockSpec((TN, FW), lambda i: (i, 0)),
        out_shape=jax.ShapeDtypeStruct((NP, FW), jnp.float32),
    )(nodes, mask, W1, W23)


def _sc_body(ei_hbm, m_hbm,
             z_hbm, c_hbm, d_hbm,
             s_sh, g_sh,
             idx_s0, idx_s1, idx_d0, idx_d1, idx_d2, rows0, rows1, ones,
             si0, si1, sd0, sd1, sd2, sg0, sg1, ss0, ss1, sgg0, sgg1):
    tid = lax.axis_index("s")
    nbase = tid * ROWS_PT
    ebase = tid * EDG_PT
    idx_s = (idx_s0, idx_s1)
    idx_d = (idx_d0, idx_d1, idx_d2)
    rows = (rows0, rows1)
    si = (si0, si1)
    sd = (sd0, sd1, sd2)
    sg = (sg0, sg1)
    ss = (ss0, ss1)
    sgg = (sgg0, sgg1)

    def _zero_row(i, _):
        rows0[i, :] = jnp.zeros((FW,), jnp.float32)
        return 0
    lax.fori_loop(0, ROWS_PT, _zero_row, 0)

    def _one_row(i, _):
        ones[i, :] = jnp.ones((FW,), jnp.float32)
        return 0
    lax.fori_loop(0, CH, _one_row, 0)

    zsrc = rows0.at[pl.ds(0, ROWS_PT)]
    pltpu.sync_copy(zsrc, s_sh.at[pl.ds(nbase, ROWS_PT)])
    pltpu.sync_copy(zsrc, g_sh.at[pl.ds(nbase, ROWS_PT)])
    plsc.subcore_barrier()

    def _pipelined_round(gather_src, acc_sh, do_deg):
        def _issue_idx(k):
            base = ebase + k * CH
            pltpu.async_copy(ei_hbm.at[0, pl.ds(base, CH)],
                             idx_s[k % 2], si[k % 2])
            pltpu.async_copy(ei_hbm.at[1, pl.ds(base, CH)],
                             idx_d[k % 3], sd[k % 3])

        def _wait_scatters(k):
            b = k % 2
            pltpu.make_async_copy(rows[b], acc_sh.at[idx_d[k % 3]],
                                  ss[b]).wait()
            if do_deg:
                pltpu.make_async_copy(ones, g_sh.at[idx_d[k % 3]],
                                      sgg[b]).wait()

        _issue_idx(0)
        _issue_idx(1)
        pltpu.make_async_copy(ei_hbm.at[0, pl.ds(ebase, CH)],
                              idx_s[0], si[0]).wait()
        pltpu.async_copy(gather_src.at[idx_s[0]], rows[0], sg[0])
        for k in range(NCH):
            b = k % 2
            nb = 1 - b
            if k + 1 < NCH:
                if k >= 1:
                    _wait_scatters(k - 1)
                pltpu.make_async_copy(ei_hbm.at[0, pl.ds(ebase, CH)],
                                      idx_s[nb], si[nb]).wait()
                pltpu.async_copy(gather_src.at[idx_s[nb]], rows[nb], sg[nb])
            pltpu.make_async_copy(gather_src.at[idx_s[b]],
                                  rows[b], sg[b]).wait()
            pltpu.make_async_copy(ei_hbm.at[1, pl.ds(ebase, CH)],
                                  idx_d[k % 3], sd[k % 3]).wait()
            pltpu.async_copy(rows[b], acc_sh.at[idx_d[k % 3]], ss[b],
                             add=True)
            if do_deg:
                pltpu.async_copy(ones, g_sh.at[idx_d[k % 3]], sgg[b],
                                 add=True)
            if k + 2 < NCH:
                _issue_idx(k + 2)
        _wait_scatters(NCH - 2)
        _wait_scatters(NCH - 1)

    _pipelined_round(m_hbm, s_sh, True)
    plsc.subcore_barrier()

    sbuf = rows0.at[pl.ds(0, ROWS_PT)]
    dbuf = ones.at[pl.ds(0, ROWS_PT)]
    pltpu.sync_copy(s_sh.at[pl.ds(nbase, ROWS_PT)], sbuf)
    pltpu.sync_copy(g_sh.at[pl.ds(nbase, ROWS_PT)], dbuf)

    def _div(i, _):
        rows0[i, :] = rows0[i, :] / jnp.maximum(ones[i, :], 1.0)
        rows1[i, :] = jnp.zeros((FW,), jnp.float32)
        return 0
    lax.fori_loop(0, ROWS_PT, _div, 0)
    pltpu.sync_copy(sbuf, z_hbm.at[pl.ds(nbase, ROWS_PT)])
    pltpu.sync_copy(dbuf, d_hbm.at[pl.ds(nbase, ROWS_PT)])
    pltpu.sync_copy(rows1.at[pl.ds(0, ROWS_PT)], s_sh.at[pl.ds(nbase, ROWS_PT)])
    plsc.subcore_barrier()

    _pipelined_round(z_hbm, s_sh, False)
    plsc.subcore_barrier()

    cbuf = rows0.at[pl.ds(0, ROWS_PT)]
    pltpu.sync_copy(s_sh.at[pl.ds(nbase, ROWS_PT)], cbuf)
    pltpu.sync_copy(cbuf, c_hbm.at[pl.ds(nbase, ROWS_PT)])


def _sc_passes(edge_index, M):
    mesh = plsc.VectorSubcoreMesh(
        core_axis_name="c", subcore_axis_name="s", num_cores=1)
    f = functools.partial(
        pl.kernel,
        out_type=(
            jax.ShapeDtypeStruct((NP, FW), jnp.float32),
            jax.ShapeDtypeStruct((NP, FW), jnp.float32),
            jax.ShapeDtypeStruct((NP, FW), jnp.float32),
        ),
        mesh=mesh,
        scratch_types=[
            pltpu.VMEM_SHARED((NP, FW), jnp.float32),
            pltpu.VMEM_SHARED((NP, FW), jnp.float32),
            pltpu.VMEM((CH,), jnp.int32),
            pltpu.VMEM((CH,), jnp.int32),
            pltpu.VMEM((CH,), jnp.int32),
            pltpu.VMEM((CH,), jnp.int32),
            pltpu.VMEM((CH,), jnp.int32),
            pltpu.VMEM((CH, FW), jnp.float32),
            pltpu.VMEM((CH, FW), jnp.float32),
            pltpu.VMEM((CH, FW), jnp.float32),
            pltpu.SemaphoreType.DMA,
            pltpu.SemaphoreType.DMA,
            pltpu.SemaphoreType.DMA,
            pltpu.SemaphoreType.DMA,
            pltpu.SemaphoreType.DMA,
            pltpu.SemaphoreType.DMA,
            pltpu.SemaphoreType.DMA,
            pltpu.SemaphoreType.DMA,
            pltpu.SemaphoreType.DMA,
            pltpu.SemaphoreType.DMA,
            pltpu.SemaphoreType.DMA,
        ],
        compiler_params=pltpu.CompilerParams(use_tc_tiling_on_sc=False),
    )(_sc_body)
    return f(edge_index, M)


def _epilogue_body(c_ref, z_ref, g_ref, sc_ref, b_ref, out_ref):
    deg = jnp.maximum(g_ref[...][:, :1], 1.0)
    x = c_ref[...][:, :K] / deg + z_ref[...][:, K:]
    mu = jnp.mean(x, axis=-1, keepdims=True)
    var = jnp.mean((x - mu) ** 2, axis=-1, keepdims=True)
    y = (x - mu) * lax.rsqrt(var + 1e-5) * sc_ref[...] + b_ref[...]
    out_ref[...] = jnp.maximum(y, 0.0)


def _epilogue(C, Z, deg, ln_scale, ln_bias):
    TN = 2000
    return pl.pallas_call(
        _epilogue_body,
        grid=(N // TN,),
        in_specs=[
            pl.BlockSpec((TN, FW), lambda i: (i, 0)),
            pl.BlockSpec((TN, FW), lambda i: (i, 0)),
            pl.BlockSpec((TN, FW), lambda i: (i, 0)),
            pl.BlockSpec((1, K), lambda i: (0, 0)),
            pl.BlockSpec((1, K), lambda i: (0, 0)),
        ],
        out_specs=pl.BlockSpec((TN, K), lambda i: (i, 0)),
        out_shape=jax.ShapeDtypeStruct((N, K), jnp.float32),
    )(C, Z, deg, ln_scale, ln_bias)


def kernel(nodes, mask, none_selected, edge_index, W1, W2, W3, ln_scale, ln_bias):
    del none_selected
    W23 = jnp.concatenate([W2, W3], axis=1)
    M = _project(nodes, mask, W1, W23)
    Z, C, deg = _sc_passes(edge_index, M)
    out = _epilogue(C, Z, deg,
                    ln_scale.reshape(1, K), ln_bias.reshape(1, K))
    return out


# speedup vs baseline: 28.1410x; 1.0986x over previous
import functools

import jax
import jax.numpy as jnp
from jax import lax
from jax.experimental import pallas as pl
from jax.experimental.pallas import tpu as pltpu
from jax.experimental.pallas import tpu_sc as plsc

N = 10000
E = 320000
D = 128
K = 8
FW = 2 * K

NSUB = 16
NCORE = 2
NP = 10240
ROWS_PT = NP // NSUB
CH = 2000
EDG_PT = E // (NSUB * NCORE)
NCH = EDG_PT // CH


def _project_body(nodes_ref, mask_ref, w1_ref, w23_ref, out_ref):
    t = nodes_ref[...] * mask_ref[...]
    h = jnp.dot(t, w1_ref[...], preferred_element_type=jnp.float32)
    out_ref[...] = jnp.dot(h, w23_ref[...], preferred_element_type=jnp.float32)


def _project(nodes, mask, W1, W23):
    TN = 2000
    return pl.pallas_call(
        _project_body,
        grid=(N // TN,),
        in_specs=[
            pl.BlockSpec((TN, D), lambda i: (i, 0)),
            pl.BlockSpec((TN, 1), lambda i: (i, 0)),
            pl.BlockSpec((D, D), lambda i: (0, 0)),
            pl.BlockSpec((D, FW), lambda i: (0, 0)),
        ],
        out_specs=pl.BlockSpec((TN, FW), lambda i: (i, 0)),
        out_shape=jax.ShapeDtypeStruct((NP, FW), jnp.float32),
    )(nodes, mask, W1, W23)


def _pipelined_round(ei_hbm, gather_src, acc_sh, g_sh, ebase,
                     idx_s, idx_d, rows, ones, si, sd, sg, ss, sgg):
    do_deg = g_sh is not None

    def _issue_idx(k):
        base = ebase + k * CH
        pltpu.async_copy(ei_hbm.at[0, pl.ds(base, CH)], idx_s[k % 2],
                         si[k % 2])
        pltpu.async_copy(ei_hbm.at[1, pl.ds(base, CH)], idx_d[k % 3],
                         sd[k % 3])

    def _wait_scatters(k):
        b = k % 2
        pltpu.make_async_copy(rows[b], acc_sh.at[idx_d[k % 3]], ss[b]).wait()
        if do_deg:
            pltpu.make_async_copy(ones, g_sh.at[idx_d[k % 3]], sgg[b]).wait()

    _issue_idx(0)
    _issue_idx(1)
    pltpu.make_async_copy(ei_hbm.at[0, pl.ds(ebase, CH)],
                          idx_s[0], si[0]).wait()
    pltpu.async_copy(gather_src.at[idx_s[0]], rows[0], sg[0])
    for k in range(NCH):
        b = k % 2
        nb = 1 - b
        if k + 1 < NCH:
            if k >= 1:
                _wait_scatters(k - 1)
            pltpu.make_async_copy(ei_hbm.at[0, pl.ds(ebase, CH)],
                                  idx_s[nb], si[nb]).wait()
            pltpu.async_copy(gather_src.at[idx_s[nb]], rows[nb], sg[nb])
        pltpu.make_async_copy(gather_src.at[idx_s[b]], rows[b], sg[b]).wait()
        pltpu.make_async_copy(ei_hbm.at[1, pl.ds(ebase, CH)],
                              idx_d[k % 3], sd[k % 3]).wait()
        pltpu.async_copy(rows[b], acc_sh.at[idx_d[k % 3]], ss[b], add=True)
        if do_deg:
            pltpu.async_copy(ones, g_sh.at[idx_d[k % 3]], sgg[b], add=True)
        if k + 2 < NCH:
            _issue_idx(k + 2)
    _wait_scatters(NCH - 2)
    _wait_scatters(NCH - 1)


def _sc_round1_body(ei_hbm, m_hbm,
                    s_part, d_part,
                    s_sh, g_sh,
                    idx_s0, idx_s1, idx_d0, idx_d1, idx_d2,
                    rows0, rows1, ones,
                    si0, si1, sd0, sd1, sd2, sg0, sg1, ss0, ss1, sgg0, sgg1):
    cid = lax.axis_index("c")
    sid = lax.axis_index("s")
    nbase = sid * ROWS_PT
    ebase = (cid * NSUB + sid) * EDG_PT

    def _zero_row(i, _):
        rows0[i, :] = jnp.zeros((FW,), jnp.float32)
        return 0
    lax.fori_loop(0, ROWS_PT, _zero_row, 0)

    def _one_row(i, _):
        ones[i, :] = jnp.ones((FW,), jnp.float32)
        return 0
    lax.fori_loop(0, CH, _one_row, 0)

    zsrc = rows0.at[pl.ds(0, ROWS_PT)]
    pltpu.sync_copy(zsrc, s_sh.at[pl.ds(nbase, ROWS_PT)])
    pltpu.sync_copy(zsrc, g_sh.at[pl.ds(nbase, ROWS_PT)])
    plsc.subcore_barrier()

    _pipelined_round(ei_hbm, m_hbm, s_sh, g_sh, ebase,
                     (idx_s0, idx_s1), (idx_d0, idx_d1, idx_d2),
                     (rows0, rows1), ones,
                     (si0, si1), (sd0, sd1, sd2), (sg0, sg1),
                     (ss0, ss1), (sgg0, sgg1))
    plsc.subcore_barrier()

    buf = rows0.at[pl.ds(0, ROWS_PT)]
    pltpu.sync_copy(s_sh.at[pl.ds(nbase, ROWS_PT)], buf)
    pltpu.sync_copy(buf, s_part.at[cid, pl.ds(nbase, ROWS_PT)])
    pltpu.sync_copy(g_sh.at[pl.ds(nbase, ROWS_PT)], buf)
    pltpu.sync_copy(buf, d_part.at[cid, pl.ds(nbase, ROWS_PT)])


def _sc_round1(edge_index, M):
    mesh = plsc.VectorSubcoreMesh(
        core_axis_name="c", subcore_axis_name="s", num_cores=NCORE)
    f = functools.partial(
        pl.kernel,
        out_type=(
            jax.ShapeDtypeStruct((NCORE, NP, FW), jnp.float32),
            jax.ShapeDtypeStruct((NCORE, NP, FW), jnp.float32),
        ),
        mesh=mesh,
        scratch_types=[
            pltpu.VMEM_SHARED((NP, FW), jnp.float32),
            pltpu.VMEM_SHARED((NP, FW), jnp.float32),
            pltpu.VMEM((CH,), jnp.int32),
            pltpu.VMEM((CH,), jnp.int32),
            pltpu.VMEM((CH,), jnp.int32),
            pltpu.VMEM((CH,), jnp.int32),
            pltpu.VMEM((CH,), jnp.int32),
            pltpu.VMEM((CH, FW), jnp.float32),
            pltpu.VMEM((CH, FW), jnp.float32),
            pltpu.VMEM((CH, FW), jnp.float32),
        ] + [pltpu.SemaphoreType.DMA] * 11,
        compiler_params=pltpu.CompilerParams(use_tc_tiling_on_sc=False),
    )(_sc_round1_body)
    return f(edge_index, M)


def _sc_round2_body(ei_hbm, s_part, d_part,
                    z_hbm, c_part, d_hbm,
                    z_sh, c_sh,
                    idx_s0, idx_s1, idx_d0, idx_d1, idx_d2,
                    rows0, rows1,
                    si0, si1, sd0, sd1, sd2, sg0, sg1, ss0, ss1):
    cid = lax.axis_index("c")
    sid = lax.axis_index("s")
    nbase = sid * ROWS_PT
    ebase = (cid * NSUB + sid) * EDG_PT

    a = rows0.at[pl.ds(0, ROWS_PT)]
    b = rows0.at[pl.ds(ROWS_PT, ROWS_PT)]
    c = rows1.at[pl.ds(0, ROWS_PT)]
    d = rows1.at[pl.ds(ROWS_PT, ROWS_PT)]
    pltpu.sync_copy(s_part.at[0, pl.ds(nbase, ROWS_PT)], a)
    pltpu.sync_copy(s_part.at[1, pl.ds(nbase, ROWS_PT)], b)
    pltpu.sync_copy(d_part.at[0, pl.ds(nbase, ROWS_PT)], c)
    pltpu.sync_copy(d_part.at[1, pl.ds(nbase, ROWS_PT)], d)

    def _combine(i, _):
        dsum = rows1[i, :] + rows1[ROWS_PT + i, :]
        s = rows0[i, :] + rows0[ROWS_PT + i, :]
        rows0[i, :] = s / jnp.maximum(dsum, 1.0)
        rows0[ROWS_PT + i, :] = dsum
        rows1[i, :] = jnp.zeros((FW,), jnp.float32)
        return 0
    lax.fori_loop(0, ROWS_PT, _combine, 0)

    pltpu.sync_copy(a, z_sh.at[pl.ds(nbase, ROWS_PT)])
    pltpu.sync_copy(c, c_sh.at[pl.ds(nbase, ROWS_PT)])

    @pl.when(cid == 0)
    def _():
        pltpu.sync_copy(a, z_hbm.at[pl.ds(nbase, ROWS_PT)])
        pltpu.sync_copy(b, d_hbm.at[pl.ds(nbase, ROWS_PT)])
    plsc.subcore_barrier()

    _pipelined_round(ei_hbm, z_sh, c_sh, None, ebase,
                     (idx_s0, idx_s1), (idx_d0, idx_d1, idx_d2),
                     (rows0, rows1), None,
                     (si0, si1), (sd0, sd1, sd2), (sg0, sg1),
                     (ss0, ss1), None)
    plsc.subcore_barrier()

    buf = rows0.at[pl.ds(0, ROWS_PT)]
    pltpu.sync_copy(c_sh.at[pl.ds(nbase, ROWS_PT)], buf)
    pltpu.sync_copy(buf, c_part.at[cid, pl.ds(nbase, ROWS_PT)])


def _sc_round2(edge_index, s_part, d_part):
    mesh = plsc.VectorSubcoreMesh(
        core_axis_name="c", subcore_axis_name="s", num_cores=NCORE)
    f = functools.partial(
        pl.kernel,
        out_type=(
            jax.ShapeDtypeStruct((NP, FW), jnp.float32),
            jax.ShapeDtypeStruct((NCORE, NP, FW), jnp.float32),
            jax.ShapeDtypeStruct((NP, FW), jnp.float32),
        ),
        mesh=mesh,
        scratch_types=[
            pltpu.VMEM_SHARED((NP, FW), jnp.float32),
            pltpu.VMEM_SHARED((NP, FW), jnp.float32),
            pltpu.VMEM((CH,), jnp.int32),
            pltpu.VMEM((CH,), jnp.int32),
            pltpu.VMEM((CH,), jnp.int32),
            pltpu.VMEM((CH,), jnp.int32),
            pltpu.VMEM((CH,), jnp.int32),
            pltpu.VMEM((CH, FW), jnp.float32),
            pltpu.VMEM((CH, FW), jnp.float32),
        ] + [pltpu.SemaphoreType.DMA] * 9,
        compiler_params=pltpu.CompilerParams(use_tc_tiling_on_sc=False),
    )(_sc_round2_body)
    return f(edge_index, s_part, d_part)


def _epilogue_body(c_ref, z_ref, g_ref, sc_ref, b_ref, out_ref):
    cp = c_ref[...]
    csum = cp[0] + cp[1]
    deg = jnp.maximum(g_ref[...][:, :1], 1.0)
    x = csum[:, :K] / deg + z_ref[...][:, K:]
    mu = jnp.mean(x, axis=-1, keepdims=True)
    var = jnp.mean((x - mu) ** 2, axis=-1, keepdims=True)
    y = (x - mu) * lax.rsqrt(var + 1e-5) * sc_ref[...] + b_ref[...]
    out_ref[...] = jnp.maximum(y, 0.0)


def _epilogue(C, Z, deg, ln_scale, ln_bias):
    TN = 2000
    return pl.pallas_call(
        _epilogue_body,
        grid=(N // TN,),
        in_specs=[
            pl.BlockSpec((NCORE, TN, FW), lambda i: (0, i, 0)),
            pl.BlockSpec((TN, FW), lambda i: (i, 0)),
            pl.BlockSpec((TN, FW), lambda i: (i, 0)),
            pl.BlockSpec((1, K), lambda i: (0, 0)),
            pl.BlockSpec((1, K), lambda i: (0, 0)),
        ],
        out_specs=pl.BlockSpec((TN, K), lambda i: (i, 0)),
        out_shape=jax.ShapeDtypeStruct((N, K), jnp.float32),
    )(C, Z, deg, ln_scale, ln_bias)


def kernel(nodes, mask, none_selected, edge_index, W1, W2, W3, ln_scale, ln_bias):
    del none_selected
    W23 = jnp.concatenate([W2, W3], axis=1)
    M = _project(nodes, mask, W1, W23)
    S_part, D_part = _sc_round1(edge_index, M)
    Z, C_part, deg = _sc_round2(edge_index, S_part, D_part)
    out = _epilogue(C_part, Z, deg,
                    ln_scale.reshape(1, K), ln_bias.reshape(1, K))
    return out


# speedup vs baseline: 29.0313x; 1.0316x over previous
import functools

import jax
import jax.numpy as jnp
from jax import lax
from jax.experimental import pallas as pl
from jax.experimental.pallas import tpu as pltpu
from jax.experimental.pallas import tpu_sc as plsc

N = 10000
E = 320000
D = 128
K = 8
FW = 2 * K

NSUB = 16
NCORE = 2
NP = 10240
ROWS_PT = NP // NSUB
CH = 2000
EDG_PT = E // (NSUB * NCORE)
NCH = EDG_PT // CH


def _project_body(nodes_ref, mask_ref, w1_ref, w23_ref, out_ref):
    t = nodes_ref[...] * mask_ref[...]
    h = jnp.dot(t, w1_ref[...], preferred_element_type=jnp.float32)
    out_ref[...] = jnp.dot(h, w23_ref[...], preferred_element_type=jnp.float32)


def _project(nodes, mask, W1, W23):
    TN = 2000
    return pl.pallas_call(
        _project_body,
        grid=(N // TN,),
        in_specs=[
            pl.BlockSpec((TN, D), lambda i: (i, 0)),
            pl.BlockSpec((TN, 1), lambda i: (i, 0)),
            pl.BlockSpec((D, D), lambda i: (0, 0)),
            pl.BlockSpec((D, FW), lambda i: (0, 0)),
        ],
        out_specs=pl.BlockSpec((TN, FW), lambda i: (i, 0)),
        out_shape=jax.ShapeDtypeStruct((NP, FW), jnp.float32),
    )(nodes, mask, W1, W23)


def _pipelined_round(ei_hbm, gather_src, acc_sh, g_sh, ebase,
                     idx_s, idx_d, rows, ones, si, sd, sg, ss, sgg):
    do_deg = g_sh is not None

    def _issue_idx(k):
        base = ebase + k * CH
        pltpu.async_copy(ei_hbm.at[0, pl.ds(base, CH)], idx_s[k % 2],
                         si[k % 2])
        pltpu.async_copy(ei_hbm.at[1, pl.ds(base, CH)], idx_d[k % 3],
                         sd[k % 3])

    def _wait_scatters(k):
        b = k % 2
        pltpu.make_async_copy(rows[b], acc_sh.at[idx_d[k % 3]], ss[b]).wait()
        if do_deg:
            pltpu.make_async_copy(ones, g_sh.at[idx_d[k % 3]], sgg[b]).wait()

    _issue_idx(0)
    _issue_idx(1)
    pltpu.make_async_copy(ei_hbm.at[0, pl.ds(ebase, CH)],
                          idx_s[0], si[0]).wait()
    pltpu.async_copy(gather_src.at[idx_s[0]], rows[0], sg[0])
    for k in range(NCH):
        b = k % 2
        nb = 1 - b
        if k + 1 < NCH:
            if k >= 1:
                _wait_scatters(k - 1)
            pltpu.make_async_copy(ei_hbm.at[0, pl.ds(ebase, CH)],
                                  idx_s[nb], si[nb]).wait()
            pltpu.async_copy(gather_src.at[idx_s[nb]], rows[nb], sg[nb])
        pltpu.make_async_copy(gather_src.at[idx_s[b]], rows[b], sg[b]).wait()
        pltpu.make_async_copy(ei_hbm.at[1, pl.ds(ebase, CH)],
                              idx_d[k % 3], sd[k % 3]).wait()
        pltpu.async_copy(rows[b], acc_sh.at[idx_d[k % 3]], ss[b], add=True)
        if do_deg:
            pltpu.async_copy(ones, g_sh.at[idx_d[k % 3]], sgg[b], add=True)
        if k + 2 < NCH:
            _issue_idx(k + 2)
    _wait_scatters(NCH - 2)
    _wait_scatters(NCH - 1)


def _sc_round1_body(ei_hbm, m_hbm,
                    s_part, d_part,
                    s_sh, g_sh,
                    idx_s0, idx_s1, idx_d0, idx_d1, idx_d2,
                    rows0, rows1, ones,
                    si0, si1, sd0, sd1, sd2, sg0, sg1, ss0, ss1, sgg0, sgg1):
    cid = lax.axis_index("c")
    sid = lax.axis_index("s")
    nbase = sid * ROWS_PT
    ebase = (cid * NSUB + sid) * EDG_PT

    def _zero_row(i, _):
        rows0[i, :] = jnp.zeros((FW,), jnp.float32)
        return 0
    lax.fori_loop(0, ROWS_PT, _zero_row, 0)

    def _one_row(i, _):
        ones[i, :] = jnp.ones((FW,), jnp.float32)
        return 0
    lax.fori_loop(0, CH, _one_row, 0)

    zsrc = rows0.at[pl.ds(0, ROWS_PT)]
    pltpu.sync_copy(zsrc, s_sh.at[pl.ds(nbase, ROWS_PT)])
    pltpu.sync_copy(zsrc, g_sh.at[pl.ds(nbase, ROWS_PT)])
    plsc.subcore_barrier()

    _pipelined_round(ei_hbm, m_hbm, s_sh, g_sh, ebase,
                     (idx_s0, idx_s1), (idx_d0, idx_d1, idx_d2),
                     (rows0, rows1), ones,
                     (si0, si1), (sd0, sd1, sd2), (sg0, sg1),
                     (ss0, ss1), (sgg0, sgg1))
    plsc.subcore_barrier()

    buf = rows0.at[pl.ds(0, ROWS_PT)]
    pltpu.sync_copy(s_sh.at[pl.ds(nbase, ROWS_PT)], buf)
    pltpu.sync_copy(buf, s_part.at[cid, pl.ds(nbase, ROWS_PT)])
    pltpu.sync_copy(g_sh.at[pl.ds(nbase, ROWS_PT)], buf)
    pltpu.sync_copy(buf, d_part.at[cid, pl.ds(nbase, ROWS_PT)])


def _sc_round1(edge_index, M):
    mesh = plsc.VectorSubcoreMesh(
        core_axis_name="c", subcore_axis_name="s", num_cores=NCORE)
    f = functools.partial(
        pl.kernel,
        out_type=(
            jax.ShapeDtypeStruct((NCORE, NP, FW), jnp.float32),
            jax.ShapeDtypeStruct((NCORE, NP, FW), jnp.float32),
        ),
        mesh=mesh,
        scratch_types=[
            pltpu.VMEM_SHARED((NP, FW), jnp.float32),
            pltpu.VMEM_SHARED((NP, FW), jnp.float32),
            pltpu.VMEM((CH,), jnp.int32),
            pltpu.VMEM((CH,), jnp.int32),
            pltpu.VMEM((CH,), jnp.int32),
            pltpu.VMEM((CH,), jnp.int32),
            pltpu.VMEM((CH,), jnp.int32),
            pltpu.VMEM((CH, FW), jnp.float32),
            pltpu.VMEM((CH, FW), jnp.float32),
            pltpu.VMEM((CH, FW), jnp.float32),
        ] + [pltpu.SemaphoreType.DMA] * 11,
        compiler_params=pltpu.CompilerParams(use_tc_tiling_on_sc=False),
    )(_sc_round1_body)
    return f(edge_index, M)


def _sc_round2_body(ei_hbm, s_part, d_part,
                    z_hbm, c_part, d_hbm,
                    z_sh, c_sh,
                    idx_s0, idx_s1, idx_d0, idx_d1, idx_d2,
                    rows0, rows1,
                    si0, si1, sd0, sd1, sd2, sg0, sg1, ss0, ss1):
    cid = lax.axis_index("c")
    sid = lax.axis_index("s")
    nbase = sid * ROWS_PT
    ebase = (cid * NSUB + sid) * EDG_PT

    a = rows0.at[pl.ds(0, ROWS_PT)]
    b = rows0.at[pl.ds(ROWS_PT, ROWS_PT)]
    c = rows1.at[pl.ds(0, ROWS_PT)]
    d = rows1.at[pl.ds(ROWS_PT, ROWS_PT)]
    pltpu.sync_copy(s_part.at[0, pl.ds(nbase, ROWS_PT)], a)
    pltpu.sync_copy(s_part.at[1, pl.ds(nbase, ROWS_PT)], b)
    pltpu.sync_copy(d_part.at[0, pl.ds(nbase, ROWS_PT)], c)
    pltpu.sync_copy(d_part.at[1, pl.ds(nbase, ROWS_PT)], d)

    def _combine(i, _):
        dsum = rows1[i, :] + rows1[ROWS_PT + i, :]
        s = rows0[i, :] + rows0[ROWS_PT + i, :]
        rows0[i, :] = s / jnp.maximum(dsum, 1.0)
        rows0[ROWS_PT + i, :] = dsum
        rows1[i, :] = jnp.zeros((FW,), jnp.float32)
        return 0
    lax.fori_loop(0, ROWS_PT, _combine, 0)

    pltpu.sync_copy(a, z_sh.at[pl.ds(nbase, ROWS_PT)])
    pltpu.sync_copy(c, c_sh.at[pl.ds(nbase, ROWS_PT)])

    @pl.when(cid == 0)
    def _():
        pltpu.sync_copy(a, z_hbm.at[pl.ds(nbase, ROWS_PT)])
        pltpu.sync_copy(b, d_hbm.at[pl.ds(nbase, ROWS_PT)])
    plsc.subcore_barrier()

    _pipelined_round(ei_hbm, z_sh, c_sh, None, ebase,
                     (idx_s0, idx_s1), (idx_d0, idx_d1, idx_d2),
                     (rows0, rows1), None,
                     (si0, si1), (sd0, sd1, sd2), (sg0, sg1),
                     (ss0, ss1), None)
    plsc.subcore_barrier()

    buf = rows0.at[pl.ds(0, ROWS_PT)]
    pltpu.sync_copy(c_sh.at[pl.ds(nbase, ROWS_PT)], buf)
    pltpu.sync_copy(buf, c_part.at[cid, pl.ds(nbase, ROWS_PT)])


def _sc_round2(edge_index, s_part, d_part):
    mesh = plsc.VectorSubcoreMesh(
        core_axis_name="c", subcore_axis_name="s", num_cores=NCORE)
    f = functools.partial(
        pl.kernel,
        out_type=(
            jax.ShapeDtypeStruct((NP, FW), jnp.float32),
            jax.ShapeDtypeStruct((NCORE, NP, FW), jnp.float32),
            jax.ShapeDtypeStruct((NP, FW), jnp.float32),
        ),
        mesh=mesh,
        scratch_types=[
            pltpu.VMEM_SHARED((NP, FW), jnp.float32),
            pltpu.VMEM_SHARED((NP, FW), jnp.float32),
            pltpu.VMEM((CH,), jnp.int32),
            pltpu.VMEM((CH,), jnp.int32),
            pltpu.VMEM((CH,), jnp.int32),
            pltpu.VMEM((CH,), jnp.int32),
            pltpu.VMEM((CH,), jnp.int32),
            pltpu.VMEM((CH, FW), jnp.float32),
            pltpu.VMEM((CH, FW), jnp.float32),
        ] + [pltpu.SemaphoreType.DMA] * 9,
        compiler_params=pltpu.CompilerParams(use_tc_tiling_on_sc=False),
    )(_sc_round2_body)
    return f(edge_index, s_part, d_part)


RC = NP // (NSUB * NCORE)
OC = RC // 2


def _sc_ln_body(c_part, d_hbm, z_hbm, lnsb_hbm,
                outp,
                cbuf0, cbuf1, dbuf, zbuf, obuf, lnv):
    cid = lax.axis_index("c")
    sid = lax.axis_index("s")
    w = cid * NSUB + sid
    nbase = w * RC
    obase = w * OC
    pltpu.sync_copy(c_part.at[0, pl.ds(nbase, RC)], cbuf0)
    pltpu.sync_copy(c_part.at[1, pl.ds(nbase, RC)], cbuf1)
    pltpu.sync_copy(d_hbm.at[pl.ds(nbase, RC)], dbuf)
    pltpu.sync_copy(z_hbm.at[pl.ds(nbase, RC)], zbuf)
    pltpu.sync_copy(lnsb_hbm, lnv)

    lane = lax.iota(jnp.int32, FW)
    mfirst = lane < K
    shfwd = jnp.minimum(lane + K, FW - 1)
    shback = jnp.maximum(lane - K, 0)
    lnvv = lnv[...]
    scalev = lnvv
    biasv = lnvv.at[shfwd].get(mode="promise_in_bounds")
    sh8 = jnp.bitwise_xor(lane, 8)
    sh4 = jnp.bitwise_xor(lane, 4)
    sh2 = jnp.bitwise_xor(lane, 2)
    sh1 = jnp.bitwise_xor(lane, 1)

    def _lanesum(v):
        for p in (sh8, sh4, sh2, sh1):
            v = v + v.at[p].get(mode="promise_in_bounds")
        return v

    def _row(r):
        cv = cbuf0[r, :] + cbuf1[r, :]
        degv = jnp.maximum(dbuf[r, :], 1.0)
        zv = zbuf[r, :]
        z3 = zv.at[shfwd].get(mode="promise_in_bounds")
        x = cv / degv + z3
        xm = jnp.where(mfirst, x, 0.0)
        mu = _lanesum(xm) * (1.0 / K)
        dx = jnp.where(mfirst, x - mu, 0.0)
        var = _lanesum(dx * dx) * (1.0 / K)
        tv = var + 1e-5
        ti = lax.bitcast_convert_type(tv, jnp.int32)
        y = lax.bitcast_convert_type(
            jnp.int32(0x5F3759DF) - lax.shift_right_logical(ti, 1),
            jnp.float32)
        for _ in range(3):
            y = y * (1.5 - 0.5 * tv * y * y)
        return jnp.maximum(dx * y * scalev + biasv, 0.0)

    def _pair(i, _):
        o0 = _row(2 * i)
        o1 = _row(2 * i + 1)
        o1s = o1.at[shback].get(mode="promise_in_bounds")
        obuf[i, :] = jnp.where(mfirst, o0, o1s)
        return 0
    lax.fori_loop(0, OC, _pair, 0)
    pltpu.sync_copy(obuf, outp.at[pl.ds(obase, OC)])


def _sc_ln(C_part, deg, Z, lnsb):
    mesh = plsc.VectorSubcoreMesh(
        core_axis_name="c", subcore_axis_name="s", num_cores=NCORE)
    f = functools.partial(
        pl.kernel,
        out_type=jax.ShapeDtypeStruct((NP // 2, FW), jnp.float32),
        mesh=mesh,
        scratch_types=[
            pltpu.VMEM((RC, FW), jnp.float32),
            pltpu.VMEM((RC, FW), jnp.float32),
            pltpu.VMEM((RC, FW), jnp.float32),
            pltpu.VMEM((RC, FW), jnp.float32),
            pltpu.VMEM((OC, FW), jnp.float32),
            pltpu.VMEM((FW,), jnp.float32),
        ],
        compiler_params=pltpu.CompilerParams(use_tc_tiling_on_sc=False),
    )(_sc_ln_body)
    return f(C_part, deg, Z, lnsb)


def kernel(nodes, mask, none_selected, edge_index, W1, W2, W3, ln_scale, ln_bias):
    del none_selected
    W23 = jnp.concatenate([W2, W3], axis=1)
    M = _project(nodes, mask, W1, W23)
    S_part, D_part = _sc_round1(edge_index, M)
    Z, C_part, deg = _sc_round2(edge_index, S_part, D_part)
    lnsb = jnp.concatenate([ln_scale, ln_bias])
    packed = _sc_ln(C_part, deg, Z, lnsb)
    return packed.reshape(NP, K)[:N]
